# baseline (speedup 1.0000x reference)
import functools
import jax
import jax.numpy as jnp
from jax.experimental import pallas as pl
from jax.experimental.pallas import tpu as pltpu

EPS_BN = 1e-05


def _square_distance(src, dst):
    dist = -2.0 * jnp.matmul(src, jnp.transpose(dst, (0, 2, 1)))
    dist = dist + jnp.sum(src ** 2, axis=-1)[:, :, None]
    dist = dist + jnp.sum(dst ** 2, axis=-1)[:, None, :]
    return dist


def _index_points(points, idx):
    B = points.shape[0]
    bidx = jnp.arange(B)[:, None, None]
    return points[bidx, idx]


def _farthest_point_sample(xyz, npoint):
    B, N, _ = xyz.shape
    bidx = jnp.arange(B)
    def step(carry, _):
        distance, farthest = carry
        centroid_xyz = xyz[bidx, farthest][:, None, :]
        dist = jnp.sum((xyz - centroid_xyz) ** 2, axis=-1)
        distance = jnp.minimum(distance, dist)
        new_far = jnp.argmax(distance, axis=1).astype(jnp.int32)
        return (distance, new_far), farthest
    init = (jnp.full((B, N), 1e10, dtype=xyz.dtype), jnp.zeros((B,), dtype=jnp.int32))
    _, cent = jax.lax.scan(step, init, None, length=npoint)
    return jnp.transpose(cent, (1, 0))


def _query_ball_point(radius, nsample, xyz, new_xyz):
    B, N, _ = xyz.shape
    S = new_xyz.shape[1]
    sqrdists = _square_distance(new_xyz, xyz)
    gi = jnp.broadcast_to(jnp.arange(N, dtype=jnp.int32)[None, None, :], (B, S, N))
    gi = jnp.where(sqrdists > radius ** 2, N, gi)
    gi = jnp.sort(gi, axis=-1)[:, :, :nsample]
    invalid = gi == N
    first = gi[:, :, 0]
    first = jnp.where(first == N, 0, first)
    gi = jnp.where(invalid, first[:, :, None], gi)
    gi = jnp.clip(gi, 0, N - 1)
    return gi


def _conv1d(x, w, b):
    return jnp.einsum('oi,bin->bon', w, x) + b[None, :, None]


def _conv2d(x, w, b):
    return jnp.einsum('oi,bihw->bohw', w, x) + b[None, :, None, None]


def _bn_eval(x, g, bt):
    scale = g / jnp.sqrt(1.0 + EPS_BN)
    if x.ndim == 3:
        return x * scale[None, :, None] + bt[None, :, None]
    return x * scale[None, :, None, None] + bt[None, :, None, None]


def _sa_forward(xyz, points, npoint, radius, nsample, p):
    B, _, N = xyz.shape
    xyz_t = jnp.transpose(xyz, (0, 2, 1))
    cidx = _farthest_point_sample(xyz_t, npoint)
    new_xyz = xyz_t[jnp.arange(B)[:, None], cidx]
    gi = _query_ball_point(radius, nsample, xyz_t, new_xyz)
    grouped_xyz = _index_points(xyz_t, gi) - new_xyz[:, :, None, :]
    if points is not None:
        grouped_points = _index_points(jnp.transpose(points, (0, 2, 1)), gi)
        new_points = jnp.concatenate([grouped_xyz, grouped_points], axis=-1)
    else:
        new_points = grouped_xyz
    new_points = jnp.transpose(new_points, (0, 3, 2, 1))
    for (w, b, g, bt) in p:
        new_points = jax.nn.relu(_conv2d(new_points, w, b))
        new_points = jax.nn.relu(_bn_eval(new_points, g, bt))
    new_points = jnp.max(new_points, axis=2)
    return jnp.transpose(new_xyz, (0, 2, 1)), new_points


def _fp_forward(xyz1, xyz2, points1, points2, p):
    B, _, N = xyz1.shape
    dists = _square_distance(jnp.transpose(xyz1, (0, 2, 1)), jnp.transpose(xyz2, (0, 2, 1)))
    idx = jnp.argsort(dists, axis=-1)[:, :, :3]
    d3 = jnp.take_along_axis(dists, idx, axis=-1)
    dist_recip = 1.0 / (d3 + 1e-08)
    norm = jnp.sum(dist_recip, axis=2, keepdims=True)
    weight = dist_recip / norm
    pts2 = jnp.transpose(points2, (0, 2, 1))
    interp = jnp.sum(_index_points(pts2, idx) * weight[..., None], axis=2)
    interp = jnp.transpose(interp, (0, 2, 1))
    if points1 is not None:
        x = jnp.concatenate([interp, points1], axis=1)
    else:
        x = interp
    for (w, b, g, bt) in p:
        x = _conv1d(x, w, b)
        if g is not None:
            x = _bn_eval(x, g, bt)
        x = jax.nn.relu(x)
    return x


def _head_body(x_ref, w1_ref, b1_ref, s1_ref, t1_ref, w2_ref, b2_ref, o_ref):
    x = x_ref[...]
    y = jax.lax.dot_general(x, w1_ref[...], (((1,), (1,)), ((), ())),
                            preferred_element_type=jnp.float32)
    y = y + b1_ref[...]
    y = y * s1_ref[...] + t1_ref[...]
    y = jnp.maximum(y, 0.0)
    z = jax.lax.dot_general(y, w2_ref[...], (((1,), (1,)), ((), ())),
                            preferred_element_type=jnp.float32)
    o_ref[...] = z + b2_ref[...]


def _head_pallas(l0_points, params):
    B, C, N = l0_points.shape
    x = jnp.transpose(l0_points, (0, 2, 1)).reshape(B * N, C)
    w1 = params['conv1_w']
    b1 = params['conv1_b'][None, :]
    s1 = (params['bn1_g'] / jnp.sqrt(1.0 + EPS_BN))[None, :]
    t1 = params['bn1_b'][None, :]
    w2 = params['conv2_w']
    O = w2.shape[0]
    OP = 16
    w2p = jnp.zeros((OP, C), w2.dtype).at[:O].set(w2)
    b2p = jnp.zeros((1, OP), w2.dtype).at[0, :O].set(params['conv2_b'])
    BLK = 512
    grid = (B * N // BLK,)
    out = pl.pallas_call(
        _head_body,
        grid=grid,
        in_specs=[
            pl.BlockSpec((BLK, C), lambda i: (i, 0)),
            pl.BlockSpec((C, C), lambda i: (0, 0)),
            pl.BlockSpec((1, C), lambda i: (0, 0)),
            pl.BlockSpec((1, C), lambda i: (0, 0)),
            pl.BlockSpec((1, C), lambda i: (0, 0)),
            pl.BlockSpec((OP, C), lambda i: (0, 0)),
            pl.BlockSpec((1, OP), lambda i: (0, 0)),
        ],
        out_specs=pl.BlockSpec((BLK, OP), lambda i: (i, 0)),
        out_shape=jax.ShapeDtypeStruct((B * N, OP), jnp.float32),
    )(x, w1, b1, s1, t1, w2p, b2p)
    return out[:, :O].reshape(B, N, O)


def kernel(xyz, params):
    l1_xyz, l1_points = _sa_forward(xyz, None, 1024, 0.1, 32, params['sa1'])
    l2_xyz, l2_points = _sa_forward(l1_xyz, l1_points, 256, 0.2, 32, params['sa2'])
    l1_points = _fp_forward(l1_xyz, l2_xyz, l1_points, l2_points, params['fp1'])
    l0_points = _fp_forward(xyz, l1_xyz, None, l1_points, params['fp2'])
    return _head_pallas(l0_points, params)


# speedup vs baseline: 7.5848x; 7.5848x over previous
import functools
import jax
import jax.numpy as jnp
from jax import lax
from jax.experimental import pallas as pl
from jax.experimental.pallas import tpu as pltpu
from jax.experimental.pallas import tpu_sc as plsc

EPS_BN = 1e-05
F32 = jnp.float32
I32 = jnp.int32


def _sqdist_in_kernel(s, d2):
    m = jax.lax.dot_general(s, d2, (((1,), (0,)), ((), ())),
                            preferred_element_type=F32)
    dist = m * -2.0
    dist = dist + jnp.sum(s * s, axis=1, keepdims=True)
    dist = dist + jnp.sum(d2 * d2, axis=0, keepdims=True)
    return dist


def _index_points(points, idx):
    B = points.shape[0]
    bidx = jnp.arange(B)[:, None, None]
    return points[bidx, idx]


def _ballq_body(c_ref, x2_ref, out_ref, *, r2, nsample):
    c = c_ref[0]
    x2 = x2_ref[0]
    SBLK = c.shape[0]
    N = x2.shape[1]
    NC = N // 128
    dist = _sqdist_in_kernel(c, x2)
    mask = (dist <= r2).astype(F32).reshape(SBLK, NC, 128)
    tri = (jax.lax.broadcasted_iota(I32, (128, 128), 0)
           <= jax.lax.broadcasted_iota(I32, (128, 128), 1)).astype(F32)
    pre = jax.lax.dot_general(mask, tri, (((2,), (0,)), ((), ())),
                              preferred_element_type=F32)
    tchunk = jnp.sum(mask, axis=2)
    tri_s = (jax.lax.broadcasted_iota(I32, (NC, NC), 0)
             < jax.lax.broadcasted_iota(I32, (NC, NC), 1)).astype(F32)
    off = jax.lax.dot_general(tchunk, tri_s, (((1,), (0,)), ((), ())),
                              preferred_element_type=F32)
    cnt = pre + off[:, :, None]
    tot = jnp.sum(tchunk, axis=1, keepdims=True)
    colid = (jax.lax.broadcasted_iota(I32, (SBLK, NC, 128), 1) * 128
             + jax.lax.broadcasted_iota(I32, (SBLK, NC, 128), 2))
    maskb = mask > 0.0
    cols = []
    first = None
    for k in range(nsample):
        sel = jnp.logical_and(maskb, cnt == jnp.float32(k + 1))
        idx = jnp.sum(jnp.sum(jnp.where(sel, colid, 0), axis=2), axis=1)[:, None]
        if k == 0:
            first = idx
            cols.append(idx)
        else:
            valid = tot >= jnp.float32(k + 1)
            cols.append(jnp.where(valid, idx, first))
    out_ref[0] = jnp.concatenate(cols, axis=1) + pl.program_id(0) * N


def _query_ball_point(radius, nsample, xyz2, new_xyz_t):
    B, _, N = xyz2.shape
    S = new_xyz_t.shape[1]
    SBLK = 128
    out = pl.pallas_call(
        functools.partial(_ballq_body, r2=radius ** 2, nsample=nsample),
        grid=(B, S // SBLK),
        in_specs=[
            pl.BlockSpec((1, SBLK, 3), lambda b, i: (b, i, 0)),
            pl.BlockSpec((1, 3, N), lambda b, i: (b, 0, 0)),
        ],
        out_specs=pl.BlockSpec((1, SBLK, nsample), lambda b, i: (b, i, 0)),
        out_shape=jax.ShapeDtypeStruct((B, S, nsample), I32),
    )(new_xyz_t, xyz2)
    return out


def _fps_body(xs_ref, ys_ref, zs_ref, out_ref, cx_ref, cy_ref, cz_ref, *, npoint):
    B, N = xs_ref.shape
    xs = xs_ref[...]
    ys = ys_ref[...]
    zs = zs_ref[...]
    iota = jax.lax.broadcasted_iota(I32, (B, N), 1)
    eye_i = (jax.lax.broadcasted_iota(I32, (B, B), 0)
             == jax.lax.broadcasted_iota(I32, (B, B), 1))
    eye = eye_i.astype(I32)
    eye_f = eye_i.astype(F32)

    def step(t, carry):
        distance, far = carry
        out_ref[pl.ds(t, 1), :] = jnp.sum(far * eye, axis=0, keepdims=True)
        oh = (iota == far).astype(F32)
        cx = jnp.sum(xs * oh, axis=1, keepdims=True)
        cy = jnp.sum(ys * oh, axis=1, keepdims=True)
        cz = jnp.sum(zs * oh, axis=1, keepdims=True)
        cx_ref[pl.ds(t, 1), :] = jnp.sum(cx * eye_f, axis=0, keepdims=True)
        cy_ref[pl.ds(t, 1), :] = jnp.sum(cy * eye_f, axis=0, keepdims=True)
        cz_ref[pl.ds(t, 1), :] = jnp.sum(cz * eye_f, axis=0, keepdims=True)
        dx = xs - cx
        dy = ys - cy
        dz = zs - cz
        d = dx * dx + dy * dy + dz * dz
        distance = jnp.minimum(distance, d)
        m = jnp.max(distance, axis=1, keepdims=True)
        far_new = jnp.min(jnp.where(distance == m, iota, N), axis=1, keepdims=True)
        return (distance, far_new)

    jax.lax.fori_loop(
        0, npoint, step,
        (jnp.full((B, N), 1e10, F32), jnp.zeros((B, 1), I32)))


def _fps_pallas(xyz_t, npoint):
    B, N, _ = xyz_t.shape
    xs = xyz_t[..., 0]
    ys = xyz_t[..., 1]
    zs = xyz_t[..., 2]
    outs = pl.pallas_call(
        functools.partial(_fps_body, npoint=npoint),
        grid=(1,),
        in_specs=[pl.BlockSpec((B, N), lambda i: (0, 0))] * 3,
        out_specs=[pl.BlockSpec((npoint, B), lambda i: (0, 0))] * 4,
        out_shape=[jax.ShapeDtypeStruct((npoint, B), I32)]
        + [jax.ShapeDtypeStruct((npoint, B), F32)] * 3,
    )(xs, ys, zs)
    cidx = jnp.transpose(outs[0], (1, 0))
    new_xyz = jnp.stack(
        [jnp.transpose(o, (1, 0)) for o in outs[1:]], axis=-1)
    return cidx, new_xyz


def _sc_gather(table, idx):
    R, D = table.shape
    T = idx.shape[0]
    NW = 32
    CH = 128
    per_w = T // NW
    n_ch = per_w // CH
    mesh = plsc.VectorSubcoreMesh(core_axis_name="c", subcore_axis_name="s")

    @functools.partial(
        pl.kernel, mesh=mesh,
        out_type=jax.ShapeDtypeStruct((T, D), F32),
        scratch_types=[
            pltpu.VMEM((CH,), I32),
            pltpu.VMEM((CH, D), F32),
            pltpu.SemaphoreType.DMA,
        ],
    )
    def k(table_hbm, idx_hbm, out_hbm, idx_v, rows_v, sem):
        wid = lax.axis_index("s") * 2 + lax.axis_index("c")
        base = wid * per_w

        def body(j, carry):
            off = base + j * CH
            pltpu.sync_copy(idx_hbm.at[pl.ds(off, CH)], idx_v)
            pltpu.async_copy(table_hbm.at[idx_v], rows_v, sem).wait()
            pltpu.sync_copy(rows_v, out_hbm.at[pl.ds(off, CH)])
            return carry

        lax.fori_loop(0, n_ch, body, 0)

    return k(table, idx)


def _sa_mlp_body(*refs, nlayers):
    x_ref = refs[0]
    c_ref = refs[1]
    out_ref = refs[-1]
    x = x_ref[...] - c_ref[...][:, None, :]
    for l in range(nlayers):
        w, b, s, t = refs[2 + 4 * l: 6 + 4 * l]
        x = jax.lax.dot_general(x, w[...], (((2,), (1,)), ((), ())),
                                preferred_element_type=F32)
        x = jnp.maximum(x + b[...], 0.0)
        x = jnp.maximum(x * s[...] + t[...], 0.0)
    out_ref[...] = jnp.max(x, axis=1)


def _sa_mlp_pallas(grouped, cent_pad, p):
    BS, K, CP = grouped.shape
    SBLK = 128
    grid = (BS // SBLK,)
    args = [grouped, cent_pad]
    in_specs = [pl.BlockSpec((SBLK, K, CP), lambda i: (i, 0, 0)),
                pl.BlockSpec((SBLK, CP), lambda i: (i, 0))]
    for li, (w, b, g, bt) in enumerate(p):
        O, CI = w.shape
        if li == 0 and CI != CP:
            w = jnp.zeros((O, CP), F32).at[:, :CI].set(w)
        scale = (g / jnp.sqrt(1.0 + EPS_BN)).reshape(1, 1, O)
        args += [w, b.reshape(1, 1, O), scale, bt.reshape(1, 1, O)]
        in_specs += [
            pl.BlockSpec(w.shape, lambda i: (0, 0)),
            pl.BlockSpec((1, 1, O), lambda i: (0, 0, 0)),
            pl.BlockSpec((1, 1, O), lambda i: (0, 0, 0)),
            pl.BlockSpec((1, 1, O), lambda i: (0, 0, 0)),
        ]
    CO = p[-1][0].shape[0]
    out = pl.pallas_call(
        functools.partial(_sa_mlp_body, nlayers=len(p)),
        grid=grid,
        in_specs=in_specs,
        out_specs=pl.BlockSpec((SBLK, CO), lambda i: (i, 0)),
        out_shape=jax.ShapeDtypeStruct((BS, CO), F32),
    )(*args)
    return out


def _sa_forward(xyz_t, xyz_c, feats_t, npoint, radius, nsample, p):
    B, N, _ = xyz_t.shape
    cidx, new_xyz = _fps_pallas(xyz_t, npoint)
    gi = _query_ball_point(radius, nsample, xyz_c, new_xyz)
    if feats_t is not None:
        CI = 3 + feats_t.shape[2]
        table = jnp.concatenate([xyz_t, feats_t], axis=2).reshape(B * N, CI)
    else:
        CI = 3
        table = xyz_t.reshape(B * N, CI)
    CP = ((CI + 127) // 128) * 128
    if CP != CI:
        table = jnp.concatenate(
            [table, jnp.zeros((B * N, CP - CI), F32)], axis=1)
    rows = _sc_gather(table, gi.reshape(-1))
    cent_pad = jnp.concatenate(
        [new_xyz, jnp.zeros((B, npoint, CP - 3), F32)], axis=2).reshape(-1, CP)
    new_points = _sa_mlp_pallas(
        rows.reshape(B * npoint, nsample, CP), cent_pad, p)
    return new_xyz, new_points.reshape(B, npoint, -1)


def _fp_sel_body(dist_ref, idx_ref, d3_ref):
    dist = dist_ref[0]
    BLK, N2 = dist.shape
    iota = jax.lax.broadcasted_iota(I32, (BLK, N2), 1)
    BIG = jnp.float32(3.4e38)
    idxs = []
    mins = []
    d_cur = dist
    for _ in range(3):
        mn = jnp.min(d_cur, axis=1, keepdims=True)
        ix = jnp.min(jnp.where(d_cur == mn, iota, N2), axis=1, keepdims=True)
        idxs.append(ix)
        mins.append(mn)
        d_cur = jnp.where(iota == ix, BIG, d_cur)
    idx_ref[0] = jnp.concatenate(idxs, axis=1)
    d3_ref[0] = jnp.concatenate(mins, axis=1)


def _fp_select(dists, BLK):
    B, N1, N2 = dists.shape
    idx, d3 = pl.pallas_call(
        _fp_sel_body,
        grid=(B, N1 // BLK),
        in_specs=[pl.BlockSpec((1, BLK, N2), lambda b, i: (b, i, 0))],
        out_specs=[pl.BlockSpec((1, BLK, 3), lambda b, i: (b, i, 0))] * 2,
        out_shape=[jax.ShapeDtypeStruct((B, N1, 3), I32),
                   jax.ShapeDtypeStruct((B, N1, 3), F32)],
    )(dists)
    return idx, d3


def _fp_body(*refs, nlayers, has_p1, has_head, N2):
    i = 0
    idx_ref = refs[i]; i += 1
    w_ref = refs[i]; i += 1
    p2_ref = refs[i]; i += 1
    p1_ref = None
    if has_p1:
        p1_ref = refs[i]; i += 1
    mlp_refs = refs[i:-1]
    out_ref = refs[-1]

    idx = idx_ref[0]
    wgt = w_ref[0]
    BLK = idx.shape[0]
    iota = jax.lax.broadcasted_iota(I32, (BLK, N2), 1)
    M = ((iota == idx[:, 0:1]).astype(F32) * wgt[:, 0:1]
         + (iota == idx[:, 1:2]).astype(F32) * wgt[:, 1:2]
         + (iota == idx[:, 2:3]).astype(F32) * wgt[:, 2:3])
    interp = jax.lax.dot_general(M, p2_ref[0], (((1,), (0,)), ((), ())),
                                 preferred_element_type=F32,
                                 precision=jax.lax.Precision.HIGHEST)
    if has_p1:
        x = jnp.concatenate([interp, p1_ref[0]], axis=1)
    else:
        x = interp

    j = 0
    for l in range(nlayers):
        w, b, sc, sh = mlp_refs[j:j + 4]; j += 4
        x = jax.lax.dot_general(x, w[...], (((1,), (1,)), ((), ())),
                                preferred_element_type=F32) + b[...]
        if sc is not None:
            x = x * sc[...] + sh[...]
        x = jnp.maximum(x, 0.0)
    if has_head:
        hw1, hb1, hs1, ht1, hw2, hb2 = mlp_refs[j:j + 6]
        x = jax.lax.dot_general(x, hw1[...], (((1,), (1,)), ((), ())),
                                preferred_element_type=F32) + hb1[...]
        x = jnp.maximum(x * hs1[...] + ht1[...], 0.0)
        x = jax.lax.dot_general(x, hw2[...], (((1,), (1,)), ((), ())),
                                preferred_element_type=F32) + hb2[...]
    out_ref[0] = x


def _fp_pallas(xyz1_c, xyz2_c, pts2_t, pts1_t, p, head, BLK):
    B = xyz1_c.shape[0]
    N1 = xyz1_c.shape[2]
    N2 = xyz2_c.shape[2]
    C2 = pts2_t.shape[2]
    src = jnp.transpose(xyz1_c, (0, 2, 1))
    dst = jnp.transpose(xyz2_c, (0, 2, 1))
    dists = -2.0 * jnp.matmul(src, jnp.transpose(dst, (0, 2, 1)))
    dists = dists + jnp.sum(src ** 2, axis=-1)[:, :, None]
    dists = dists + jnp.sum(dst ** 2, axis=-1)[:, None, :]
    idx, d3 = _fp_select(dists, BLK)
    dist_recip = 1.0 / (d3 + 1e-08)
    norm = jnp.sum(dist_recip, axis=2, keepdims=True)
    weight = dist_recip / norm

    args = [idx, weight, pts2_t]
    in_specs = [
        pl.BlockSpec((1, BLK, 3), lambda b, i: (b, i, 0)),
        pl.BlockSpec((1, BLK, 3), lambda b, i: (b, i, 0)),
        pl.BlockSpec((1, N2, C2), lambda b, i: (b, 0, 0)),
    ]
    if pts1_t is not None:
        C1 = pts1_t.shape[2]
        args.append(pts1_t)
        in_specs.append(pl.BlockSpec((1, BLK, C1), lambda b, i: (b, i, 0)))

    mlp_has_sc = []
    for (w, b, g, bt) in p:
        O = w.shape[0]
        args += [w, b.reshape(1, O)]
        in_specs += [pl.BlockSpec(w.shape, lambda bb, i: (0, 0)),
                     pl.BlockSpec((1, O), lambda bb, i: (0, 0))]
        if g is not None:
            args += [(g / jnp.sqrt(1.0 + EPS_BN)).reshape(1, O),
                     bt.reshape(1, O)]
            in_specs += [pl.BlockSpec((1, O), lambda bb, i: (0, 0))] * 2
            mlp_has_sc.append(True)
        else:
            mlp_has_sc.append(False)

    if head is not None:
        hw1 = head['conv1_w']
        C = hw1.shape[0]
        hs1 = (head['bn1_g'] / jnp.sqrt(1.0 + EPS_BN)).reshape(1, C)
        hw2 = head['conv2_w']
        O2 = hw2.shape[0]
        OP = 16
        hw2p = jnp.zeros((OP, C), F32).at[:O2].set(hw2)
        hb2p = jnp.zeros((1, OP), F32).at[0, :O2].set(head['conv2_b'])
        args += [hw1, head['conv1_b'].reshape(1, C), hs1,
                 head['bn1_b'].reshape(1, C), hw2p, hb2p]
        in_specs += [
            pl.BlockSpec((C, C), lambda bb, i: (0, 0)),
            pl.BlockSpec((1, C), lambda bb, i: (0, 0)),
            pl.BlockSpec((1, C), lambda bb, i: (0, 0)),
            pl.BlockSpec((1, C), lambda bb, i: (0, 0)),
            pl.BlockSpec((OP, C), lambda bb, i: (0, 0)),
            pl.BlockSpec((1, OP), lambda bb, i: (0, 0)),
        ]
        CO = OP
    else:
        CO = p[-1][0].shape[0]

    def body(*refs):
        _fp_body_dispatch(refs, len(p), pts1_t is not None, head is not None,
                          N2, mlp_has_sc)

    out = pl.pallas_call(
        body,
        grid=(B, N1 // BLK),
        in_specs=in_specs,
        out_specs=pl.BlockSpec((1, BLK, CO), lambda b, i: (b, i, 0)),
        out_shape=jax.ShapeDtypeStruct((B, N1, CO), F32),
    )(*args)
    return out


def _fp_body_dispatch(refs, nlayers, has_p1, has_head, N2, mlp_has_sc):
    i = 0
    base = [refs[i]]; i += 1
    base.append(refs[i]); i += 1
    base.append(refs[i]); i += 1
    if has_p1:
        base.append(refs[i]); i += 1
    mlp = []
    for l in range(nlayers):
        w = refs[i]; b = refs[i + 1]; i += 2
        if mlp_has_sc[l]:
            sc = refs[i]; sh = refs[i + 1]; i += 2
        else:
            sc = None; sh = None
        mlp += [w, b, sc, sh]
    rest = list(refs[i:])
    _fp_body(*(base + mlp + rest), nlayers=nlayers, has_p1=has_p1,
             has_head=has_head, N2=N2)


def kernel(xyz, params):
    xyz_t = jnp.transpose(xyz, (0, 2, 1))
    l1_xyz, l1_pts = _sa_forward(xyz_t, xyz, None, 1024, 0.1, 32, params['sa1'])
    l1_xyz_c = jnp.transpose(l1_xyz, (0, 2, 1))
    l2_xyz, l2_pts = _sa_forward(l1_xyz, l1_xyz_c, l1_pts, 256, 0.2, 32,
                                 params['sa2'])
    l2_xyz_c = jnp.transpose(l2_xyz, (0, 2, 1))

    fp1 = _fp_pallas(l1_xyz_c, l2_xyz_c, l2_pts, l1_pts,
                     params['fp1'], None, BLK=256)
    out = _fp_pallas(xyz, l1_xyz_c, fp1, None,
                     params['fp2'], params, BLK=512)
    return out[..., :10]
